# baseline (speedup 1.0000x reference)
import jax
import jax.numpy as jnp
from jax.experimental import pallas as pl
from jax.experimental.pallas import tpu as pltpu

_U, _I = 10000, 5000
_D, _F = 32, 8
_REC_V, _TYP_V, _RES_V = 8, 8, 16
_BU = 400
_NU = _U // _BU


def _mm_t(x, w):
    return jax.lax.dot_general(x, w, (((1,), (1,)), ((), ())),
                               preferred_element_type=jnp.float32)


def _mm_ct(x, w):
    return jax.lax.dot_general(x, w, (((0,), (0,)), ((), ())),
                               preferred_element_type=jnp.float32)


def _gcn_kernel(adj_ref, rec_idx_ref, typ_idx_ref, res_idx_ref,
                ue_ref, ie_ref, rec_w_ref, typ_w_ref, res_w_ref,
                wu_ref, bu_ref, wi_ref, bi_ref,
                uo_ref, io_ref,
                u01, acc12, i3acc, i0s):
    p = pl.program_id(0)
    r = pl.program_id(1)

    @pl.when((p == 0) & (r == 0))
    def _enrich():
        wu = wu_ref[...]
        t_rec = _mm_t(rec_w_ref[...], wu[:, _D:_D + _F])
        t_typ = _mm_t(typ_w_ref[...], wu[:, _D + _F:])
        oh_rec = (rec_idx_ref[...] == jax.lax.broadcasted_iota(
            jnp.int32, (_REC_V, _U), 0)).astype(jnp.float32)
        oh_typ = (typ_idx_ref[...] == jax.lax.broadcasted_iota(
            jnp.int32, (_TYP_V, _U), 0)).astype(jnp.float32)
        eu = (_mm_t(ue_ref[...], wu[:, :_D])
              + _mm_ct(oh_rec, t_rec) + _mm_ct(oh_typ, t_typ) + bu_ref[...])
        wi = wi_ref[...]
        t_res = _mm_t(res_w_ref[...], wi[:, _D:])
        oh_res = (res_idx_ref[...] == jax.lax.broadcasted_iota(
            jnp.int32, (_RES_V, _I), 0)).astype(jnp.float32)
        ei = (_mm_t(ie_ref[...], wi[:, :_D])
              + _mm_ct(oh_res, t_res) + bi_ref[...])
        u01[:, :_D] = eu
        i0s[...] = ei
        io_ref[...] = ei
        acc12[...] = jnp.zeros_like(acc12)
        i3acc[...] = jnp.zeros_like(i3acc)

    a = adj_ref[...]

    @pl.when(p == 0)
    def _pass1():
        u1r = jnp.dot(a, i0s[...], preferred_element_type=jnp.float32)
        u01[pl.ds(r * _BU, _BU), _D:] = u1r
        acc12[...] += _mm_ct(a, u01[pl.ds(r * _BU, _BU), :])

    @pl.when(p == 1)
    def _pass2():
        u23r = jnp.dot(a, acc12[...], preferred_element_type=jnp.float32)
        u2r = u23r[:, :_D]
        i3acc[...] += _mm_ct(a, u2r)
        u01r = u01[pl.ds(r * _BU, _BU), :]
        uo_ref[...] = (u01r[:, :_D] + u01r[:, _D:]
                       + u2r + u23r[:, _D:]) * 0.25

    @pl.when((p == 1) & (r == _NU - 1))
    def _finish():
        io_ref[...] = (io_ref[...] + acc12[:, :_D] + acc12[:, _D:]
                       + i3acc[...]) * 0.25


def _cc(shape):
    return pl.BlockSpec(shape, lambda p, r: (0,) * len(shape))


def kernel(adj, recovery_stage_idx, preferred_type_idx, resource_type_idx,
           user_emb_w, item_emb_w, recovery_emb_w, type_emb_w,
           resource_type_emb_w, user_proj_w, user_proj_b, item_proj_w,
           item_proj_b):
    rec2 = recovery_stage_idx.astype(jnp.int32).reshape(1, _U)
    typ2 = preferred_type_idx.astype(jnp.int32).reshape(1, _U)
    res2 = resource_type_idx.astype(jnp.int32).reshape(1, _I)
    bu2 = user_proj_b.reshape(1, _D)
    bi2 = item_proj_b.reshape(1, _D)

    user_out, item_out = pl.pallas_call(
        _gcn_kernel,
        grid=(2, _NU),
        in_specs=[
            pl.BlockSpec((_BU, _I), lambda p, r: (r, 0)),
            _cc((1, _U)), _cc((1, _U)), _cc((1, _I)),
            _cc((_U, _D)), _cc((_I, _D)),
            _cc((_REC_V, _F)), _cc((_TYP_V, _F)), _cc((_RES_V, _F)),
            _cc((_D, _D + 2 * _F)), _cc((1, _D)),
            _cc((_D, _D + _F)), _cc((1, _D)),
        ],
        out_specs=[pl.BlockSpec((_BU, _D), lambda p, r: (r, 0)),
                   _cc((_I, _D))],
        out_shape=[jax.ShapeDtypeStruct((_U, _D), jnp.float32),
                   jax.ShapeDtypeStruct((_I, _D), jnp.float32)],
        scratch_shapes=[
            pltpu.VMEM((_U, 2 * _D), jnp.float32),
            pltpu.VMEM((_I, 2 * _D), jnp.float32),
            pltpu.VMEM((_I, _D), jnp.float32),
            pltpu.VMEM((_I, _D), jnp.float32),
        ],
        compiler_params=pltpu.CompilerParams(
            dimension_semantics=("arbitrary", "arbitrary")),
    )(adj, rec2, typ2, res2, user_emb_w, item_emb_w,
      recovery_emb_w, type_emb_w, resource_type_emb_w,
      user_proj_w, bu2, item_proj_w, bi2)
    return (user_out, item_out)


# speedup vs baseline: 1.0093x; 1.0093x over previous
import jax
import jax.numpy as jnp
from jax.experimental import pallas as pl
from jax.experimental.pallas import tpu as pltpu

_U, _I = 10000, 5000
_D, _F = 32, 8
_REC_V, _TYP_V, _RES_V = 8, 8, 16
_BU = 400
_NU = _U // _BU


def _mm_t(x, w):
    return jax.lax.dot_general(x, w, (((1,), (1,)), ((), ())),
                               preferred_element_type=jnp.float32)


def _mm_ct(x, w):
    return jax.lax.dot_general(x, w, (((0,), (0,)), ((), ())),
                               preferred_element_type=jnp.float32)


def _gcn_kernel(adj_ref, rec_idx_ref, typ_idx_ref, res_idx_ref,
                ue_ref, ie_ref, rec_w_ref, typ_w_ref, res_w_ref,
                wu_ref, bu_ref, wi_ref, bi_ref,
                uo_ref, io_ref,
                u01, acc12, i3acc, i0s, acc12_16):
    p = pl.program_id(0)
    r = pl.program_id(1)

    @pl.when((p == 0) & (r == 0))
    def _enrich():
        wu = wu_ref[...]
        t_rec = _mm_t(rec_w_ref[...], wu[:, _D:_D + _F])
        t_typ = _mm_t(typ_w_ref[...], wu[:, _D + _F:])
        oh_rec = (rec_idx_ref[...] == jax.lax.broadcasted_iota(
            jnp.int32, (_REC_V, _U), 0)).astype(jnp.float32)
        oh_typ = (typ_idx_ref[...] == jax.lax.broadcasted_iota(
            jnp.int32, (_TYP_V, _U), 0)).astype(jnp.float32)
        eu = (_mm_t(ue_ref[...], wu[:, :_D])
              + _mm_ct(oh_rec, t_rec) + _mm_ct(oh_typ, t_typ) + bu_ref[...])
        wi = wi_ref[...]
        t_res = _mm_t(res_w_ref[...], wi[:, _D:])
        oh_res = (res_idx_ref[...] == jax.lax.broadcasted_iota(
            jnp.int32, (_RES_V, _I), 0)).astype(jnp.float32)
        ei = (_mm_t(ie_ref[...], wi[:, :_D])
              + _mm_ct(oh_res, t_res) + bi_ref[...])
        u01[:, :_D] = eu.astype(jnp.bfloat16)
        i0s[...] = ei.astype(jnp.bfloat16)
        io_ref[...] = ei
        acc12[...] = jnp.zeros_like(acc12)
        i3acc[...] = jnp.zeros_like(i3acc)

    a = adj_ref[...].astype(jnp.bfloat16)

    @pl.when(p == 0)
    def _pass1():
        u1r = jnp.dot(a, i0s[...], preferred_element_type=jnp.float32)
        u01[pl.ds(r * _BU, _BU), _D:] = u1r.astype(jnp.bfloat16)
        acc12[...] += _mm_ct(a, u01[pl.ds(r * _BU, _BU), :])

    @pl.when((p == 1) & (r == 0))
    def _snap12():
        acc12_16[...] = acc12[...].astype(jnp.bfloat16)

    @pl.when(p == 1)
    def _pass2():
        u23r = jnp.dot(a, acc12_16[...], preferred_element_type=jnp.float32)
        u2r = u23r[:, :_D]
        i3acc[...] += _mm_ct(a, u2r.astype(jnp.bfloat16))
        u01r = u01[pl.ds(r * _BU, _BU), :].astype(jnp.float32)
        uo_ref[...] = (u01r[:, :_D] + u01r[:, _D:]
                       + u2r + u23r[:, _D:]) * 0.25

    @pl.when((p == 1) & (r == _NU - 1))
    def _finish():
        io_ref[...] = (io_ref[...] + acc12[:, :_D] + acc12[:, _D:]
                       + i3acc[...]) * 0.25


def _cc(shape):
    return pl.BlockSpec(shape, lambda p, r: (0,) * len(shape))


def kernel(adj, recovery_stage_idx, preferred_type_idx, resource_type_idx,
           user_emb_w, item_emb_w, recovery_emb_w, type_emb_w,
           resource_type_emb_w, user_proj_w, user_proj_b, item_proj_w,
           item_proj_b):
    rec2 = recovery_stage_idx.astype(jnp.int32).reshape(1, _U)
    typ2 = preferred_type_idx.astype(jnp.int32).reshape(1, _U)
    res2 = resource_type_idx.astype(jnp.int32).reshape(1, _I)
    bu2 = user_proj_b.reshape(1, _D)
    bi2 = item_proj_b.reshape(1, _D)

    user_out, item_out = pl.pallas_call(
        _gcn_kernel,
        grid=(2, _NU),
        in_specs=[
            pl.BlockSpec((_BU, _I), lambda p, r: (r, 0)),
            _cc((1, _U)), _cc((1, _U)), _cc((1, _I)),
            _cc((_U, _D)), _cc((_I, _D)),
            _cc((_REC_V, _F)), _cc((_TYP_V, _F)), _cc((_RES_V, _F)),
            _cc((_D, _D + 2 * _F)), _cc((1, _D)),
            _cc((_D, _D + _F)), _cc((1, _D)),
        ],
        out_specs=[pl.BlockSpec((_BU, _D), lambda p, r: (r, 0)),
                   _cc((_I, _D))],
        out_shape=[jax.ShapeDtypeStruct((_U, _D), jnp.float32),
                   jax.ShapeDtypeStruct((_I, _D), jnp.float32)],
        scratch_shapes=[
            pltpu.VMEM((_U, 2 * _D), jnp.bfloat16),
            pltpu.VMEM((_I, 2 * _D), jnp.float32),
            pltpu.VMEM((_I, _D), jnp.float32),
            pltpu.VMEM((_I, _D), jnp.bfloat16),
            pltpu.VMEM((_I, 2 * _D), jnp.bfloat16),
        ],
        compiler_params=pltpu.CompilerParams(
            dimension_semantics=("arbitrary", "arbitrary")),
    )(adj, rec2, typ2, res2, user_emb_w, item_emb_w,
      recovery_emb_w, type_emb_w, resource_type_emb_w,
      user_proj_w, bu2, item_proj_w, bi2)
    return (user_out, item_out)


# speedup vs baseline: 1.0395x; 1.0299x over previous
import jax
import jax.numpy as jnp
from jax.experimental import pallas as pl
from jax.experimental.pallas import tpu as pltpu

_U, _I = 10000, 5000
_D, _F = 32, 8
_REC_V, _TYP_V, _RES_V = 8, 8, 16
_BU = 400
_NU = _U // _BU


def _mm_t(x, w):
    return jax.lax.dot_general(x, w, (((1,), (1,)), ((), ())),
                               preferred_element_type=jnp.float32)


def _mm_ct(x, w):
    return jax.lax.dot_general(x, w, (((0,), (0,)), ((), ())),
                               preferred_element_type=jnp.float32)


def _gcn_kernel(adj_ref, rec_idx_ref, typ_idx_ref, res_idx_ref,
                ue_ref, ie_ref, rec_w_ref, typ_w_ref, res_w_ref,
                wu_ref, bu_ref, wi_ref, bi_ref,
                uo_ref, io_ref,
                u01, rhs16, acc_all):
    p = pl.program_id(0)
    r = pl.program_id(1)

    @pl.when((p == 0) & (r == 0))
    def _enrich():
        wu = wu_ref[...]
        t_rec = _mm_t(rec_w_ref[...], wu[:, _D:_D + _F])
        t_typ = _mm_t(typ_w_ref[...], wu[:, _D + _F:])
        oh_rec = (rec_idx_ref[...] == jax.lax.broadcasted_iota(
            jnp.int32, (_REC_V, _U), 0)).astype(jnp.float32)
        oh_typ = (typ_idx_ref[...] == jax.lax.broadcasted_iota(
            jnp.int32, (_TYP_V, _U), 0)).astype(jnp.float32)
        eu = (_mm_t(ue_ref[...], wu[:, :_D])
              + _mm_ct(oh_rec, t_rec) + _mm_ct(oh_typ, t_typ) + bu_ref[...])
        wi = wi_ref[...]
        t_res = _mm_t(res_w_ref[...], wi[:, _D:])
        oh_res = (res_idx_ref[...] == jax.lax.broadcasted_iota(
            jnp.int32, (_RES_V, _I), 0)).astype(jnp.float32)
        ei = (_mm_t(ie_ref[...], wi[:, :_D])
              + _mm_ct(oh_res, t_res) + bi_ref[...])
        u01[:, :_D] = eu.astype(jnp.bfloat16)
        io_ref[...] = ei
        rhs16[:, :_D] = ei.astype(jnp.bfloat16)
        rhs16[:, _D:] = jnp.zeros((_I, _D), jnp.bfloat16)
        acc_all[...] = jnp.zeros_like(acc_all)

    @pl.when((p == 1) & (r == 0))
    def _swap_rhs():
        rhs16[...] = acc_all[:, :2 * _D].astype(jnp.bfloat16)

    a = adj_ref[...].astype(jnp.bfloat16)
    ur = jnp.dot(a, rhs16[...], preferred_element_type=jnp.float32)
    ur16 = ur.astype(jnp.bfloat16)

    @pl.when(p == 0)
    def _store_u1():
        u01[pl.ds(r * _BU, _BU), _D:] = ur16[:, :_D]

    u01r = u01[pl.ds(r * _BU, _BU), :]
    u01f = u01r.astype(jnp.float32)
    m1 = p.astype(jnp.float32)
    m0 = 1.0 - m1
    y = jnp.concatenate(
        [u01f * m0, ur[:, :_D] * m1, jnp.zeros((_BU, _D), jnp.float32)],
        axis=1).astype(jnp.bfloat16)
    acc_all[...] += _mm_ct(a, y)
    uo_ref[...] = (u01f[:, :_D] + u01f[:, _D:] + ur[:, :_D]
                   + ur[:, _D:]) * 0.25

    @pl.when((p == 1) & (r == _NU - 1))
    def _finish():
        io_ref[...] = (io_ref[...] + acc_all[:, :_D] + acc_all[:, _D:2 * _D]
                       + acc_all[:, 2 * _D:3 * _D]) * 0.25


def _cc(shape):
    return pl.BlockSpec(shape, lambda p, r: (0,) * len(shape))


def kernel(adj, recovery_stage_idx, preferred_type_idx, resource_type_idx,
           user_emb_w, item_emb_w, recovery_emb_w, type_emb_w,
           resource_type_emb_w, user_proj_w, user_proj_b, item_proj_w,
           item_proj_b):
    rec2 = recovery_stage_idx.astype(jnp.int32).reshape(1, _U)
    typ2 = preferred_type_idx.astype(jnp.int32).reshape(1, _U)
    res2 = resource_type_idx.astype(jnp.int32).reshape(1, _I)
    bu2 = user_proj_b.reshape(1, _D)
    bi2 = item_proj_b.reshape(1, _D)

    user_out, item_out = pl.pallas_call(
        _gcn_kernel,
        grid=(2, _NU),
        in_specs=[
            pl.BlockSpec((_BU, _I), lambda p, r: (r, 0)),
            _cc((1, _U)), _cc((1, _U)), _cc((1, _I)),
            _cc((_U, _D)), _cc((_I, _D)),
            _cc((_REC_V, _F)), _cc((_TYP_V, _F)), _cc((_RES_V, _F)),
            _cc((_D, _D + 2 * _F)), _cc((1, _D)),
            _cc((_D, _D + _F)), _cc((1, _D)),
        ],
        out_specs=[pl.BlockSpec((_BU, _D), lambda p, r: (r, 0)),
                   _cc((_I, _D))],
        out_shape=[jax.ShapeDtypeStruct((_U, _D), jnp.float32),
                   jax.ShapeDtypeStruct((_I, _D), jnp.float32)],
        scratch_shapes=[
            pltpu.VMEM((_U, 2 * _D), jnp.bfloat16),
            pltpu.VMEM((_I, 2 * _D), jnp.bfloat16),
            pltpu.VMEM((_I, 4 * _D), jnp.float32),
        ],
        compiler_params=pltpu.CompilerParams(
            dimension_semantics=("arbitrary", "arbitrary")),
    )(adj, rec2, typ2, res2, user_emb_w, item_emb_w,
      recovery_emb_w, type_emb_w, resource_type_emb_w,
      user_proj_w, bu2, item_proj_w, bi2)
    return (user_out, item_out)
